# baseline (speedup 1.0000x reference)
import functools

import jax
import jax.numpy as jnp
import numpy as np
from jax.experimental import pallas as pl

A_MAX = 7.25
S_MAX = 9.25

B = 128
J = 22
F = 6600
FB = 5
FCHUNK = F // FB


def _field_xy():
    x = np.linspace(0.5, 119.5, 120, dtype=np.float32)
    y = np.linspace(-0.5, 53.5, 55, dtype=np.float32)
    y[0] = -0.2
    yy, xx = np.meshgrid(y, x, indexing="ij")
    return xx.reshape(FB, FCHUNK, 1), yy.reshape(FB, FCHUNK, 1)


def _body(x_ref, y_ref, vx_ref, vy_ref, fx_ref, fy_ref, out_ref):
    x = x_ref[0]
    y = y_ref[0]
    vx = vx_ref[0]
    vy = vy_ref[0]
    fx = fx_ref[0]
    fy = fy_ref[0]
    dx = fx - x
    dy = fy - y
    d2 = dx * dx + dy * dy
    d = jnp.sqrt(d2)
    s0 = jnp.clip((dx * vx + dy * vy) / d, -S_MAX, S_MAX)
    t_lt = (S_MAX - s0) / A_MAX
    d_lt = t_lt * (s0 + S_MAX) * 0.5
    u = s0 / A_MAX
    t_alt = jnp.sqrt(u * u + 2.0 * d / A_MAX) - u
    t_lt = jnp.where(d_lt > d, t_alt, t_lt)
    d_lt = jnp.maximum(jnp.minimum(d_lt, d), 0.0)
    t_at = (d - d_lt) / S_MAX
    out_ref[0] = t_lt + t_at


@jax.jit
def _run(xp, yp, vxp, vyp, fx, fy):
    return pl.pallas_call(
        _body,
        grid=(B, FB),
        in_specs=[
            pl.BlockSpec((1, 1, J), lambda b, f: (b, 0, 0)),
            pl.BlockSpec((1, 1, J), lambda b, f: (b, 0, 0)),
            pl.BlockSpec((1, 1, J), lambda b, f: (b, 0, 0)),
            pl.BlockSpec((1, 1, J), lambda b, f: (b, 0, 0)),
            pl.BlockSpec((1, FCHUNK, 1), lambda b, f: (f, 0, 0)),
            pl.BlockSpec((1, FCHUNK, 1), lambda b, f: (f, 0, 0)),
        ],
        out_specs=pl.BlockSpec((1, FCHUNK, J), lambda b, f: (b, f, 0)),
        out_shape=jax.ShapeDtypeStruct((B, F, J), jnp.float32),
    )(xp, yp, vxp, vyp, fx, fy)


def kernel(frame):
    xp = frame[:, None, :, 1]
    yp = frame[:, None, :, 2]
    vxp = frame[:, None, :, 3]
    vyp = frame[:, None, :, 4]
    fx_np, fy_np = _field_xy()
    fx = jnp.asarray(fx_np)
    fy = jnp.asarray(fy_np)
    return _run(xp, yp, vxp, vyp, fx, fy)
